# baseline (speedup 1.0000x reference)
import functools

import jax
import jax.numpy as jnp
from jax import lax
from jax.experimental import pallas as pl
from jax.experimental.pallas import tpu as pltpu
from jax.experimental.pallas import tpu_sc as plsc

F32 = jnp.float32
NW = 32
CHUNK = 112
NROW = 128
NSUB = CHUNK // 16

B = 32
N1, E1 = 1280, 5120
N3, E3 = 4800, 76800
ACC_W = 256


def _relu(x):
    return jnp.maximum(x, 0.0)


@functools.lru_cache(maxsize=None)
def _gat_sc_build(n, n_pad, n_chunks, e_real):
    mesh = plsc.VectorSubcoreMesh(core_axis_name="c", subcore_axis_name="s")

    @functools.partial(
        pl.kernel,
        mesh=mesh,
        compiler_params=pltpu.CompilerParams(needs_layout_passes=False),
        out_type=jax.ShapeDtypeStruct((NW, n_pad, ACC_W), F32),
        scratch_types=[
            pltpu.VMEM((n_chunks, CHUNK), jnp.int32),
            pltpu.VMEM((n_chunks, CHUNK), jnp.int32),
            pltpu.VMEM((n_chunks, CHUNK), jnp.int32),
            pltpu.VMEM((n,), F32),
            pltpu.VMEM((n,), F32),
            pltpu.VMEM((CHUNK, 128), F32),
            pltpu.VMEM((NROW, ACC_W), F32),
            pltpu.VMEM((NROW,), jnp.int32),
            pltpu.VMEM((ACC_W,), F32),
            pltpu.SemaphoreType.DMA,
        ],
    )
    def gat_sc(h_hbm, as_hbm, ad_hbm, src_hbm, dst_hbm, dstp_hbm, z_hbm,
               out_hbm, src_v, dst_v, dstp_v, as_v, ad_v, rows_v, sbuf,
               dstu_v, carry_v, sem):
        cid = lax.axis_index("c")
        sid = lax.axis_index("s")
        wid = sid * 2 + cid

        pltpu.sync_copy(src_hbm.at[wid], src_v)
        pltpu.sync_copy(dst_hbm.at[wid], dst_v)
        pltpu.sync_copy(dstp_hbm.at[wid], dstp_v)
        pltpu.sync_copy(as_hbm, as_v)
        pltpu.sync_copy(ad_hbm, ad_v)

        pltpu.sync_copy(z_hbm, out_hbm.at[wid])

        lanes = lax.iota(jnp.int32, 16)
        zero16 = jnp.zeros((16,), F32)

        def zrow(r, carry):
            for f in range(144 // 16, ACC_W // 16):
                sbuf[r, pl.ds(f * 16, 16)] = zero16
            return carry
        lax.fori_loop(0, NROW, zrow, 0)
        for f in range(144 // 16):
            carry_v[pl.ds(f * 16, 16)] = zero16

        def chunk_body(j, carry_d):
            pltpu.async_copy(h_hbm.at[src_v.at[j]], rows_v, sem).wait()
            for g in range(NROW // 16):
                dstu_v[pl.ds(g * 16, 16)] = n + g * 16 + lanes
            d0 = jnp.where(carry_d < 0, n_pad - 1, carry_d)
            plsc.store_scatter(dstu_v, [lanes], jnp.full((16,), 1, jnp.int32) * d0,
                               mask=lanes == 0)
            for f in range(144 // 16):
                sbuf[0, pl.ds(f * 16, 16)] = carry_v[pl.ds(f * 16, 16)]
            p = jnp.int32(0)
            last_d = carry_d
            for c in range(NSUB):
                s16 = src_v[j, pl.ds(c * 16, 16)]
                d16 = dst_v[j, pl.ds(c * 16, 16)]
                a_s = plsc.load_gather(as_v, [s16])
                a_d = plsc.load_gather(ad_v, [jnp.minimum(d16, n - 1)])
                e = a_s + a_d
                e = jnp.where(e > 0.0, e, 0.2 * e)
                ex = jnp.exp(e)
                gid = (wid * n_chunks + j) * CHUNK + c * 16 + lanes
                ex = jnp.where(gid < e_real, ex, 0.0)
                dp16 = dstp_v[j, pl.ds(c * 16, 16)]
                if c == 0:
                    dp16 = jnp.where(lanes == 0,
                                     jnp.full((16,), 1, jnp.int32) * carry_d,
                                     dp16)
                new16 = d16 != dp16
                p16 = p + plsc.cumsum(new16.astype(jnp.int32))
                plsc.store_scatter(dstu_v, [p16], d16, mask=new16)
                keep16 = jnp.where(new16, 0.0, 1.0)
                p = p16[15]
                last_d = d16[15]
                for l in range(16):
                    r = c * 16 + l
                    sc = ex[l]
                    pp = p16[l]
                    kp = keep16[l]
                    for f in range(8):
                        val = rows_v[r, pl.ds(f * 16, 16)] * sc
                        sbuf[pp, pl.ds(f * 16, 16)] = (
                            sbuf[pp, pl.ds(f * 16, 16)] * kp + val)
                    den = jnp.where(lanes == 0, sc, 0.0)
                    sbuf[pp, pl.ds(128, 16)] = (
                        sbuf[pp, pl.ds(128, 16)] * kp + den)
            for f in range(144 // 16):
                carry_v[pl.ds(f * 16, 16)] = sbuf[p, pl.ds(f * 16, 16)]
            plsc.store_scatter(dstu_v, [jnp.full((16,), 1, jnp.int32) * p],
                               n + p + jnp.full((16,), 0, jnp.int32),
                               mask=lanes == 0)
            pltpu.sync_copy(sbuf, out_hbm.at[wid].at[dstu_v], add=True)
            return last_d

        final_d = lax.fori_loop(0, n_chunks, chunk_body, jnp.int32(-1))
        for g in range(NROW // 16):
            dstu_v[pl.ds(g * 16, 16)] = n + g * 16 + lanes
        dfin = jnp.where(final_d < 0, n_pad - 1, final_d)
        plsc.store_scatter(dstu_v, [lanes],
                           jnp.full((16,), 1, jnp.int32) * dfin,
                           mask=lanes == 0)
        for f in range(144 // 16):
            sbuf[0, pl.ds(f * 16, 16)] = carry_v[pl.ds(f * 16, 16)]
        pltpu.sync_copy(sbuf, out_hbm.at[wid].at[dstu_v], add=True)

    return gat_sc


def _pad_edges(ei, n):
    loop = jnp.arange(n, dtype=jnp.int32)
    src = jnp.concatenate([ei[0].astype(jnp.int32), loop])
    dst = jnp.concatenate([ei[1].astype(jnp.int32), loop])
    order = jnp.argsort(dst)
    src = src[order]
    dst = dst[order]
    e_real = src.shape[0]
    per = NW * CHUNK
    e_pad = ((e_real + per - 1) // per) * per
    pad = e_pad - e_real
    src = jnp.pad(src, (0, pad)).reshape(NW, e_pad // (NW * CHUNK), CHUNK)
    dst = jnp.pad(dst, (0, pad), constant_values=n)
    dstp = jnp.concatenate([jnp.full((1,), -1, jnp.int32), dst[:-1]])
    dstp = dstp.reshape(e_pad // CHUNK, CHUNK).at[:, 0].set(-1)
    nc = e_pad // (NW * CHUNK)
    dst = dst.reshape(NW, nc, CHUNK)
    dstp = dstp.reshape(NW, nc, CHUNK)
    return src, dst, dstp, e_real, nc


def _dot(a, b):
    return jnp.dot(a, b, preferred_element_type=F32)


def _prot_mlp_call(x3, dis, w00, b00, w01, b01, w02p, b02p, w4, a4s, a4d):
    blk = 480
    gi = N3 // blk

    def body(x_ref, d_ref, w00_r, b00_r, w01_r, b01_r, w02_r, b02_r,
             w4_r, a4s_r, a4d_r, ami0_r, h_r, as_r, ad_r):
        y = _relu(_dot(x_ref[...], w00_r[...]) + b00_r[...])
        y = _relu(_dot(y, w01_r[...]) + b01_r[...])
        y = _relu(_dot(y, w02_r[...]) + b02_r[...])
        col = lax.broadcasted_iota(jnp.int32, (blk, 128), 1)
        ami0 = jnp.where(col == 127, d_ref[...], y)
        ami0_r[...] = ami0
        h = _dot(ami0, w4_r[...])
        h_r[...] = h
        as_r[...] = _dot(h, a4s_r[...])
        ad_r[...] = _dot(h, a4d_r[...])

    full = lambda s: pl.BlockSpec(s, lambda i: (0,) * len(s))
    return pl.pallas_call(
        body,
        grid=(gi,),
        in_specs=[
            pl.BlockSpec((blk, 1900), lambda i: (i, 0)),
            pl.BlockSpec((blk, 1), lambda i: (i, 0)),
            full((1900, 1024)), full((1, 1024)),
            full((1024, 512)), full((1, 512)),
            full((512, 128)), full((1, 128)),
            full((128, 128)), full((128, 1)), full((128, 1)),
        ],
        out_specs=[
            pl.BlockSpec((blk, 128), lambda i: (i, 0)),
            pl.BlockSpec((blk, 128), lambda i: (i, 0)),
            pl.BlockSpec((blk, 1), lambda i: (i, 0)),
            pl.BlockSpec((blk, 1), lambda i: (i, 0)),
        ],
        out_shape=[
            jax.ShapeDtypeStruct((N3, 128), F32),
            jax.ShapeDtypeStruct((N3, 128), F32),
            jax.ShapeDtypeStruct((N3, 1), F32),
            jax.ShapeDtypeStruct((N3, 1), F32),
        ],
    )(x3, dis, w00, b00, w01, b01, w02p, b02p, w4, a4s, a4d)


def _drug_mlp_call(x1, w1, b1, w2, b2, w03, b03, wc1, a1s, a1d):
    def body(x_r, w1_r, b1_r, w2_r, b2_r, w03_r, b03_r, wc_r, as_r_w, ad_r_w,
             x10_r, h_r, as_r, ad_r):
        y = _relu(_dot(x_r[...], w1_r[...]) + b1_r[...])
        y = _relu(_dot(y, w2_r[...]) + b2_r[...])
        x10 = _dot(y, w03_r[...]) + b03_r[...]
        x10_r[...] = x10
        h = _dot(x10, wc_r[...])
        h_r[...] = h
        as_r[...] = _dot(h, as_r_w[...])
        ad_r[...] = _dot(h, ad_r_w[...])

    return pl.pallas_call(
        body,
        out_shape=[
            jax.ShapeDtypeStruct((N1, 128), F32),
            jax.ShapeDtypeStruct((N1, 128), F32),
            jax.ShapeDtypeStruct((N1, 1), F32),
            jax.ShapeDtypeStruct((N1, 1), F32),
        ],
    )(x1, w1, b1, w2, b2, w03, b03, wc1, a1s, a1d)


def _gat_epi_call(acc, bvec, run, w_next, asrc_n, adst_n):
    n = run.shape[0]
    rblk = 240 if n % 240 == 0 else 160
    gi = n // rblk
    full = lambda s: pl.BlockSpec(s, lambda i: (0,) * len(s))

    def body(acc_r, b_r, run_r, w_r, as_w, ad_w, run_o, h_o, as_o, ad_o):
        a = jnp.sum(acc_r[...], axis=0)
        num = a[:, :128]
        den = a[:, 128:129]
        x = _relu(num / (den + 1e-16) + b_r[...])
        run_o[...] = run_r[...] + x
        h = _dot(x, w_r[...])
        h_o[...] = h
        as_o[...] = _dot(h, as_w[...])
        ad_o[...] = _dot(h, ad_w[...])

    return pl.pallas_call(
        body,
        grid=(gi,),
        in_specs=[
            pl.BlockSpec((NW, rblk, ACC_W), lambda i: (0, i, 0)),
            full((1, 128)),
            pl.BlockSpec((rblk, 128), lambda i: (i, 0)),
            full((128, 128)), full((128, 1)), full((128, 1)),
        ],
        out_specs=[
            pl.BlockSpec((rblk, 128), lambda i: (i, 0)),
            pl.BlockSpec((rblk, 128), lambda i: (i, 0)),
            pl.BlockSpec((rblk, 1), lambda i: (i, 0)),
            pl.BlockSpec((rblk, 1), lambda i: (i, 0)),
        ],
        out_shape=[
            jax.ShapeDtypeStruct((n, 128), F32),
            jax.ShapeDtypeStruct((n, 128), F32),
            jax.ShapeDtypeStruct((n, 1), F32),
            jax.ShapeDtypeStruct((n, 1), F32),
        ],
    )(acc, bvec, run, w_next, asrc_n, adst_n)


def _gat_epi_last_call(acc, bvec, run):
    n = run.shape[0]
    rblk = 240 if n % 240 == 0 else 160
    gi = n // rblk
    full = lambda s: pl.BlockSpec(s, lambda i: (0,) * len(s))

    def body(acc_r, b_r, run_r, run_o):
        a = jnp.sum(acc_r[...], axis=0)
        x = _relu(a[:, :128] / (a[:, 128:129] + 1e-16) + b_r[...])
        run_o[...] = run_r[...] + x

    return pl.pallas_call(
        body,
        grid=(gi,),
        in_specs=[
            pl.BlockSpec((NW, rblk, ACC_W), lambda i: (0, i, 0)),
            full((1, 128)),
            pl.BlockSpec((rblk, 128), lambda i: (i, 0)),
        ],
        out_specs=pl.BlockSpec((rblk, 128), lambda i: (i, 0)),
        out_shape=jax.ShapeDtypeStruct((n, 128), F32),
    )(acc, bvec, run)


def _qkv_call(xg, amis, wq, bq, wk, bk, wv, bv, batch1c, amib_row):
    def body(xg_r, am_r, wq_r, bq_r, wk_r, bk_r, wv_r, bv_r, b1_r, ab_r,
             q_o, k_o, v_o, ep_o, wp_o):
        q = _dot(xg_r[...], wq_r[...]) + bq_r[...]
        q_o[...] = q
        k_o[...] = _dot(am_r[...], wk_r[...]) + bk_r[...]
        v_o[...] = _dot(am_r[...], wv_r[...]) + bv_r[...]
        scale = 1.0 / jnp.sqrt(128.0)
        ep_o[...] = _dot(q, bk_r[...].reshape(128, 1)) * scale
        gids = lax.broadcasted_iota(jnp.int32, (B, N3), 0)
        cnt = jnp.sum(jnp.where(ab_r[...] == gids, 1.0, 0.0), axis=1,
                      keepdims=True)
        maxp = jnp.max(cnt)
        pad_true = maxp - cnt
        nfree = jnp.float32(N3) - cnt
        w_pad = nfree * pad_true / jnp.maximum(nfree, 1.0)
        g2 = lax.broadcasted_iota(jnp.int32, (N1, B), 1)
        onehot = jnp.where(b1_r[...] == g2, 1.0, 0.0)
        wp_o[...] = _dot(onehot, w_pad)

    return pl.pallas_call(
        body,
        out_shape=[
            jax.ShapeDtypeStruct((N1, 128), F32),
            jax.ShapeDtypeStruct((N3, 128), F32),
            jax.ShapeDtypeStruct((N3, 128), F32),
            jax.ShapeDtypeStruct((N1, 1), F32),
            jax.ShapeDtypeStruct((N1, 1), F32),
        ],
    )(xg, amis, wq, bq, wk, bk, wv, bv, batch1c, amib_row)


def _cross_att_call(q, k, v, batch1c, amib_row, epad, wprow,
                    cafc_W, cafc_b, wv_b, ln_g, ln_b):
    blk = 128
    gi = N1 // blk

    def body(q_r, k_r, v_r, b1_r, ab_r, ep_r, wp_r, cw_r, cb_r, vb_r,
             lg_r, lb_r, out_r):
        scale = 1.0 / jnp.sqrt(128.0)
        e = lax.dot_general(q_r[...], k_r[...],
                            (((1,), (1,)), ((), ())),
                            preferred_element_type=F32) * scale
        mask = b1_r[...] == ab_r[...]
        em = jnp.where(mask, e, -1e30)
        ep = ep_r[...]
        m = jnp.maximum(jnp.max(em, axis=1, keepdims=True), ep)
        ex = jnp.exp(em - m)
        epw = wp_r[...] * jnp.exp(ep - m)
        s = jnp.sum(ex, axis=1, keepdims=True) + epw
        o = (_dot(ex, v_r[...]) + epw * vb_r[...]) / s
        o = _dot(o, cw_r[...]) + cb_r[...]
        mu = jnp.mean(o, axis=1, keepdims=True)
        var = jnp.mean((o - mu) ** 2, axis=1, keepdims=True)
        out_r[...] = (o - mu) / jnp.sqrt(var + 1e-5) * lg_r[...] + lb_r[...]

    full = lambda s: pl.BlockSpec(s, lambda i: (0,) * len(s))
    return pl.pallas_call(
        body,
        grid=(gi,),
        in_specs=[
            pl.BlockSpec((blk, 128), lambda i: (i, 0)),
            full((N3, 128)), full((N3, 128)),
            pl.BlockSpec((blk, 1), lambda i: (i, 0)),
            full((1, N3)),
            pl.BlockSpec((blk, 1), lambda i: (i, 0)),
            pl.BlockSpec((blk, 1), lambda i: (i, 0)),
            full((128, 128)), full((1, 128)), full((1, 128)),
            full((1, 128)), full((1, 128)),
        ],
        out_specs=pl.BlockSpec((blk, 128), lambda i: (i, 0)),
        out_shape=jax.ShapeDtypeStruct((N1, 128), F32),
    )(q, k, v, batch1c, amib_row, epad, wprow, cafc_W, cafc_b, wv_b,
      ln_g, ln_b)


def _node_head_call(xg, ami_att, fc0_W, fc0_b, fca_W, fca_b):
    def body(xg_r, aa_r, w_r, b_r, aw_r, ab_r, out_r):
        w = w_r[...]
        x = _relu(_dot(xg_r[...], w[:128, :]) + _dot(aa_r[...], w[128:, :])
                  + b_r[...])
        a = jnp.tanh(_dot(x, aw_r[...]) + ab_r[...])
        out_r[...] = x * a

    return pl.pallas_call(
        body,
        out_shape=jax.ShapeDtypeStruct((N1, 128), F32),
    )(xg, ami_att, fc0_W, fc0_b, fca_W, fca_b)


def _seg_reduce_call(x, batch1r):
    def body(x_r, b_r, sum_o, max_o, cnt_o):
        g = pl.program_id(0)
        msk = (b_r[...] == g).astype(F32)
        cnt_o[...] = jnp.sum(msk, axis=1, keepdims=True).reshape(1, 1, 1)
        mcol = msk.reshape(N1, 1)
        sum_o[...] = jnp.sum(x_r[...] * mcol, axis=0,
                             keepdims=True).reshape(1, 1, 128)
        max_o[...] = jnp.max(jnp.where(mcol > 0.0, x_r[...], -1e30),
                             axis=0, keepdims=True).reshape(1, 1, 128)

    full = lambda s: pl.BlockSpec(s, lambda i: (0,) * len(s))
    ssum, smax, cnt = pl.pallas_call(
        body,
        grid=(B,),
        in_specs=[full((N1, 128)), full((1, N1))],
        out_specs=[
            pl.BlockSpec((1, 1, 128), lambda i: (i, 0, 0)),
            pl.BlockSpec((1, 1, 128), lambda i: (i, 0, 0)),
            pl.BlockSpec((1, 1, 1), lambda i: (i, 0, 0)),
        ],
        out_shape=[
            jax.ShapeDtypeStruct((B, 1, 128), F32),
            jax.ShapeDtypeStruct((B, 1, 128), F32),
            jax.ShapeDtypeStruct((B, 1, 1), F32),
        ],
    )(x, batch1r)
    return ssum.reshape(B, 128), smax.reshape(B, 128), cnt.reshape(B, 1)


def _final_head_call(ssum, smax, cnt, cafc_b, ln_g, ln_b, fc0_W, fc0_b,
                     fca_W, fca_b, fc1_W, fc1_b, fc2_W, fc2_b,
                     fc3_W, fc3_b, out_W, out_b):
    def body(ss_r, sm_r, c_r, cb_r, lg_r, lb_r, w0_r, b0_r, aw_r, ab_r,
             w1_r, b1_r, w2_r, b2_r, w3_r, b3_r, wo_r, bo_r,
             out_o, h2_o):
        cb = cb_r[...]
        mu = jnp.mean(cb, axis=1, keepdims=True)
        var = jnp.mean((cb - mu) ** 2, axis=1, keepdims=True)
        pad_att = (cb - mu) / jnp.sqrt(var + 1e-5) * lg_r[...] + lb_r[...]
        pr = _relu(_dot(pad_att, w0_r[...][128:, :]) + b0_r[...])
        pr = pr * jnp.tanh(_dot(pr, aw_r[...]) + ab_r[...])
        cnt = c_r[...]
        maxl = jnp.max(cnt)
        mean_x = (ss_r[...] + (maxl - cnt) * pr) / maxl
        max_x = jnp.where(maxl > cnt, jnp.maximum(sm_r[...], pr), sm_r[...])
        xh = jnp.concatenate([mean_x, max_x], axis=1)
        h2 = _dot(xh, w1_r[...]) + b1_r[...]
        h2_o[...] = h2
        x = _relu(h2)
        x = _relu(_dot(x, w2_r[...]) + b2_r[...])
        x = _relu(_dot(x, w3_r[...]) + b3_r[...])
        out_o[...] = _dot(x, wo_r[...]) + bo_r[...]

    return pl.pallas_call(
        body,
        out_shape=[
            jax.ShapeDtypeStruct((B, 1), F32),
            jax.ShapeDtypeStruct((B, 512), F32),
        ],
    )(ssum, smax, cnt, cafc_b, ln_g, ln_b, fc0_W, fc0_b, fca_W, fca_b,
      fc1_W, fc1_b, fc2_W, fc2_b, fc3_W, fc3_b, out_W, out_b)


def _gat_chain(x0_run, h, a_s, a_d, src, dst, dstp, e_real, n_chunks, n,
               weights):
    n_pad = ((n + CHUNK + 127) // 128) * 128
    sc = _gat_sc_build(n, n_pad, n_chunks, e_real)
    zeros = jnp.zeros((n_pad, ACC_W), F32)
    run = x0_run
    for li in range(3):
        acc = sc(h, a_s.reshape(n), a_d.reshape(n), src, dst, dstp, zeros)
        b_l = weights[li][3]
        if li < 2:
            w_n, as_n, ad_n, _ = weights[li + 1]
            run, h, a_s, a_d = _gat_epi_call(
                acc, b_l.reshape(1, 128), run, w_n,
                as_n.reshape(128, 1), ad_n.reshape(128, 1))
        else:
            run = _gat_epi_last_call(acc, b_l.reshape(1, 128), run)
    return run


def kernel(x1, drug_intra, drug_edge_attr, batch1, x3, ami_intra, ami_dis,
           ami_batch, ami_dis_li, w1, b1, w2, b2,
           fc00_W, fc00_b, fc01_W, fc01_b, fc02_W, fc02_b, fc03_W, fc03_b,
           conv1_W, conv1_asrc, conv1_adst, conv1_b,
           conv2_W, conv2_asrc, conv2_adst, conv2_b,
           conv3_W, conv3_asrc, conv3_adst, conv3_b,
           conv4_W, conv4_asrc, conv4_adst, conv4_b,
           conv5_W, conv5_asrc, conv5_adst, conv5_b,
           conv6_W, conv6_asrc, conv6_adst, conv6_b,
           wq_W, wq_b, wk_W, wk_b, wv_W, wv_b, cafc_W, cafc_b,
           ln_g, ln_b, fc0_W, fc0_b, fc_att_W, fc_att_b,
           fc1_W, fc1_b, fc2_W, fc2_b, fc3_W, fc3_b, out_W, out_b):
    w02p = jnp.pad(fc02_W, ((0, 0), (0, 1)))
    b02p = jnp.pad(fc02_b, (0, 1)).reshape(1, 128)
    src3, dst3, dstp3, er3, nc3 = _pad_edges(ami_intra, N3)
    src1, dst1, dstp1, er1, nc1 = _pad_edges(drug_intra, N1)
    b1c = batch1.astype(jnp.int32).reshape(N1, 1)
    b1r = batch1.astype(jnp.int32).reshape(1, N1)
    abr = ami_batch.astype(jnp.int32).reshape(1, N3)

    ami0, h4, as4, ad4 = _prot_mlp_call(
        x3, ami_dis_li.reshape(N3, 1),
        fc00_W, fc00_b.reshape(1, 1024),
        fc01_W, fc01_b.reshape(1, 512),
        w02p, b02p,
        conv4_W, conv4_asrc.reshape(128, 1), conv4_adst.reshape(128, 1))
    amis = _gat_chain(
        ami0, h4, as4, ad4, src3, dst3, dstp3, er3, nc3, N3,
        [(conv4_W, conv4_asrc, conv4_adst, conv4_b),
         (conv5_W, conv5_asrc, conv5_adst, conv5_b),
         (conv6_W, conv6_asrc, conv6_adst, conv6_b)])

    x10, h1, as1, ad1 = _drug_mlp_call(
        x1, w1, b1.reshape(1, 128), w2, b2.reshape(1, 64),
        fc03_W, fc03_b.reshape(1, 128),
        conv1_W, conv1_asrc.reshape(128, 1), conv1_adst.reshape(128, 1))
    xg = _gat_chain(
        x10, h1, as1, ad1, src1, dst1, dstp1, er1, nc1, N1,
        [(conv1_W, conv1_asrc, conv1_adst, conv1_b),
         (conv2_W, conv2_asrc, conv2_adst, conv2_b),
         (conv3_W, conv3_asrc, conv3_adst, conv3_b)])

    q, k, v, epad, wprow = _qkv_call(
        xg, amis, wq_W, wq_b.reshape(1, 128), wk_W, wk_b.reshape(1, 128),
        wv_W, wv_b.reshape(1, 128), b1c, abr)
    ami_att = _cross_att_call(
        q, k, v, b1c, abr, epad, wprow, cafc_W, cafc_b.reshape(1, 128),
        wv_b.reshape(1, 128), ln_g.reshape(1, 128), ln_b.reshape(1, 128))

    xnode = _node_head_call(xg, ami_att, fc0_W, fc0_b.reshape(1, 128),
                            fc_att_W, fc_att_b.reshape(1, 1))
    ssum, smax, cnt = _seg_reduce_call(xnode, b1r)
    out, h2 = _final_head_call(
        ssum, smax, cnt, cafc_b.reshape(1, 128), ln_g.reshape(1, 128),
        ln_b.reshape(1, 128), fc0_W, fc0_b.reshape(1, 128),
        fc_att_W, fc_att_b.reshape(1, 1),
        fc1_W, fc1_b.reshape(1, 512), fc2_W, fc2_b.reshape(1, 256),
        fc3_W, fc3_b.reshape(1, 128), out_W, out_b.reshape(1, 1))
    return (out, h2)


# speedup vs baseline: 11.8850x; 11.8850x over previous
import functools

import jax
import jax.numpy as jnp
from jax import lax
from jax.experimental import pallas as pl
from jax.experimental.pallas import tpu as pltpu
from jax.experimental.pallas import tpu_sc as plsc

F32 = jnp.float32
NW = 32
CHUNK = 112
NROW = 128
NSUB = CHUNK // 16

B = 32
N1, E1 = 1280, 5120
N3, E3 = 4800, 76800
ACC_W = 256


def _relu(x):
    return jnp.maximum(x, 0.0)


@functools.lru_cache(maxsize=None)
def _gat_sc_build(n, n_pad, n_chunks, e_real):
    mesh = plsc.VectorSubcoreMesh(core_axis_name="c", subcore_axis_name="s")

    @functools.partial(
        pl.kernel,
        mesh=mesh,
        compiler_params=pltpu.CompilerParams(needs_layout_passes=False),
        out_type=jax.ShapeDtypeStruct((NW, n_pad, ACC_W), F32),
        scratch_types=[
            pltpu.VMEM((n_chunks, CHUNK), jnp.int32),
            pltpu.VMEM((n_chunks, CHUNK), jnp.int32),
            pltpu.VMEM((n_chunks, CHUNK), jnp.int32),
            pltpu.VMEM((n,), F32),
            pltpu.VMEM((n,), F32),
            pltpu.VMEM((CHUNK, 128), F32),
            pltpu.VMEM((NROW, ACC_W), F32),
            pltpu.VMEM((NROW,), jnp.int32),
            pltpu.VMEM((ACC_W,), F32),
            pltpu.SemaphoreType.DMA,
        ],
    )
    def gat_sc(h_hbm, as_hbm, ad_hbm, src_hbm, dst_hbm, dstp_hbm, z_hbm,
               out_hbm, src_v, dst_v, dstp_v, as_v, ad_v, rows_v, sbuf,
               dstu_v, carry_v, sem):
        cid = lax.axis_index("c")
        sid = lax.axis_index("s")
        wid = sid * 2 + cid

        pltpu.sync_copy(src_hbm.at[wid], src_v)
        pltpu.sync_copy(dst_hbm.at[wid], dst_v)
        pltpu.sync_copy(dstp_hbm.at[wid], dstp_v)
        pltpu.sync_copy(as_hbm, as_v)
        pltpu.sync_copy(ad_hbm, ad_v)


        lanes = lax.iota(jnp.int32, 16)
        zero16 = jnp.zeros((16,), F32)

        def zrow(r, carry):
            for f in range(ACC_W // 16):
                sbuf[r, pl.ds(f * 16, 16)] = zero16
            return carry
        lax.fori_loop(0, NROW, zrow, 0)
        for f in range(144 // 16):
            carry_v[pl.ds(f * 16, 16)] = zero16

        def zslab(k, carry):
            pltpu.sync_copy(sbuf, out_hbm.at[wid, pl.ds(k * NROW, NROW)])
            return carry
        lax.fori_loop(0, n_pad // NROW, zslab, 0)

        def chunk_body(j, carry_d):
            pltpu.async_copy(h_hbm.at[src_v.at[j]], rows_v, sem).wait()
            for g in range(NROW // 16):
                dstu_v[pl.ds(g * 16, 16)] = n + g * 16 + lanes
            d0 = jnp.where(carry_d < 0, n_pad - 1, carry_d)
            plsc.store_scatter(dstu_v, [lanes], jnp.full((16,), 1, jnp.int32) * d0,
                               mask=lanes == 0)
            for f in range(144 // 16):
                sbuf[0, pl.ds(f * 16, 16)] = carry_v[pl.ds(f * 16, 16)]
            p = jnp.int32(0)
            last_d = carry_d
            for c in range(NSUB):
                s16 = src_v[j, pl.ds(c * 16, 16)]
                d16 = dst_v[j, pl.ds(c * 16, 16)]
                a_s = plsc.load_gather(as_v, [s16])
                a_d = plsc.load_gather(ad_v, [jnp.minimum(d16, n - 1)])
                e = a_s + a_d
                e = jnp.where(e > 0.0, e, 0.2 * e)
                ex = jnp.exp(e)
                gid = (wid * n_chunks + j) * CHUNK + c * 16 + lanes
                ex = jnp.where(gid < e_real, ex, 0.0)
                dp16 = dstp_v[j, pl.ds(c * 16, 16)]
                if c == 0:
                    dp16 = jnp.where(lanes == 0,
                                     jnp.full((16,), 1, jnp.int32) * carry_d,
                                     dp16)
                new16 = d16 != dp16
                p16 = p + plsc.cumsum(new16.astype(jnp.int32))
                plsc.store_scatter(dstu_v, [p16], d16, mask=new16)
                keep16 = jnp.where(new16, 0.0, 1.0)
                p = p16[15]
                last_d = d16[15]
                for l in range(16):
                    r = c * 16 + l
                    sc = ex[l]
                    pp = p16[l]
                    kp = keep16[l]
                    for f in range(8):
                        val = rows_v[r, pl.ds(f * 16, 16)] * sc
                        sbuf[pp, pl.ds(f * 16, 16)] = (
                            sbuf[pp, pl.ds(f * 16, 16)] * kp + val)
                    den = jnp.where(lanes == 0, sc, 0.0)
                    sbuf[pp, pl.ds(128, 16)] = (
                        sbuf[pp, pl.ds(128, 16)] * kp + den)
            for f in range(144 // 16):
                carry_v[pl.ds(f * 16, 16)] = sbuf[p, pl.ds(f * 16, 16)]
            plsc.store_scatter(dstu_v, [jnp.full((16,), 1, jnp.int32) * p],
                               n + p + jnp.full((16,), 0, jnp.int32),
                               mask=lanes == 0)
            pltpu.sync_copy(sbuf, out_hbm.at[wid].at[dstu_v], add=True)
            return last_d

        final_d = lax.fori_loop(0, n_chunks, chunk_body, jnp.int32(-1))
        for g in range(NROW // 16):
            dstu_v[pl.ds(g * 16, 16)] = n + g * 16 + lanes
        dfin = jnp.where(final_d < 0, n_pad - 1, final_d)
        plsc.store_scatter(dstu_v, [lanes],
                           jnp.full((16,), 1, jnp.int32) * dfin,
                           mask=lanes == 0)
        for f in range(144 // 16):
            sbuf[0, pl.ds(f * 16, 16)] = carry_v[pl.ds(f * 16, 16)]
        pltpu.sync_copy(sbuf, out_hbm.at[wid].at[dstu_v], add=True)

    return gat_sc


def _pad_edges(ei, n):
    loop = jnp.arange(n, dtype=jnp.int32)
    src = jnp.concatenate([ei[0].astype(jnp.int32), loop])
    dst = jnp.concatenate([ei[1].astype(jnp.int32), loop])
    order = jnp.argsort(dst)
    src = src[order]
    dst = dst[order]
    e_real = src.shape[0]
    per = NW * CHUNK
    e_pad = ((e_real + per - 1) // per) * per
    pad = e_pad - e_real
    src = jnp.pad(src, (0, pad)).reshape(NW, e_pad // (NW * CHUNK), CHUNK)
    dst = jnp.pad(dst, (0, pad), constant_values=n)
    dstp = jnp.concatenate([jnp.full((1,), -1, jnp.int32), dst[:-1]])
    dstp = dstp.reshape(e_pad // CHUNK, CHUNK).at[:, 0].set(-1)
    nc = e_pad // (NW * CHUNK)
    dst = dst.reshape(NW, nc, CHUNK)
    dstp = dstp.reshape(NW, nc, CHUNK)
    return src, dst, dstp, e_real, nc


def _dot(a, b):
    return jnp.dot(a, b, preferred_element_type=F32)


def _prot_mlp_call(x3, dis, w00, b00, w01, b01, w02p, b02p, w4, a4s, a4d):
    blk = 480
    gi = N3 // blk

    def body(x_ref, d_ref, w00_r, b00_r, w01_r, b01_r, w02_r, b02_r,
             w4_r, a4s_r, a4d_r, ami0_r, h_r, as_r, ad_r):
        y = _relu(_dot(x_ref[...], w00_r[...]) + b00_r[...])
        y = _relu(_dot(y, w01_r[...]) + b01_r[...])
        y = _relu(_dot(y, w02_r[...]) + b02_r[...])
        col = lax.broadcasted_iota(jnp.int32, (blk, 128), 1)
        ami0 = jnp.where(col == 127, d_ref[...], y)
        ami0_r[...] = ami0
        h = _dot(ami0, w4_r[...])
        h_r[...] = h
        as_r[...] = _dot(h, a4s_r[...])
        ad_r[...] = _dot(h, a4d_r[...])

    full = lambda s: pl.BlockSpec(s, lambda i: (0,) * len(s))
    return pl.pallas_call(
        body,
        grid=(gi,),
        in_specs=[
            pl.BlockSpec((blk, 1900), lambda i: (i, 0)),
            pl.BlockSpec((blk, 1), lambda i: (i, 0)),
            full((1900, 1024)), full((1, 1024)),
            full((1024, 512)), full((1, 512)),
            full((512, 128)), full((1, 128)),
            full((128, 128)), full((128, 1)), full((128, 1)),
        ],
        out_specs=[
            pl.BlockSpec((blk, 128), lambda i: (i, 0)),
            pl.BlockSpec((blk, 128), lambda i: (i, 0)),
            pl.BlockSpec((blk, 1), lambda i: (i, 0)),
            pl.BlockSpec((blk, 1), lambda i: (i, 0)),
        ],
        out_shape=[
            jax.ShapeDtypeStruct((N3, 128), F32),
            jax.ShapeDtypeStruct((N3, 128), F32),
            jax.ShapeDtypeStruct((N3, 1), F32),
            jax.ShapeDtypeStruct((N3, 1), F32),
        ],
    )(x3, dis, w00, b00, w01, b01, w02p, b02p, w4, a4s, a4d)


def _drug_mlp_call(x1, w1, b1, w2, b2, w03, b03, wc1, a1s, a1d):
    def body(x_r, w1_r, b1_r, w2_r, b2_r, w03_r, b03_r, wc_r, as_r_w, ad_r_w,
             x10_r, h_r, as_r, ad_r):
        y = _relu(_dot(x_r[...], w1_r[...]) + b1_r[...])
        y = _relu(_dot(y, w2_r[...]) + b2_r[...])
        x10 = _dot(y, w03_r[...]) + b03_r[...]
        x10_r[...] = x10
        h = _dot(x10, wc_r[...])
        h_r[...] = h
        as_r[...] = _dot(h, as_r_w[...])
        ad_r[...] = _dot(h, ad_r_w[...])

    return pl.pallas_call(
        body,
        out_shape=[
            jax.ShapeDtypeStruct((N1, 128), F32),
            jax.ShapeDtypeStruct((N1, 128), F32),
            jax.ShapeDtypeStruct((N1, 1), F32),
            jax.ShapeDtypeStruct((N1, 1), F32),
        ],
    )(x1, w1, b1, w2, b2, w03, b03, wc1, a1s, a1d)


def _gat_epi_call(acc, bvec, run, w_next, asrc_n, adst_n):
    n = run.shape[0]
    rblk = 240 if n % 240 == 0 else 160
    gi = n // rblk
    full = lambda s: pl.BlockSpec(s, lambda i: (0,) * len(s))

    def body(acc_r, b_r, run_r, w_r, as_w, ad_w, run_o, h_o, as_o, ad_o):
        a = jnp.sum(acc_r[...], axis=0)
        num = a[:, :128]
        den = a[:, 128:129]
        x = _relu(num / (den + 1e-16) + b_r[...])
        run_o[...] = run_r[...] + x
        h = _dot(x, w_r[...])
        h_o[...] = h
        as_o[...] = _dot(h, as_w[...])
        ad_o[...] = _dot(h, ad_w[...])

    return pl.pallas_call(
        body,
        grid=(gi,),
        in_specs=[
            pl.BlockSpec((NW, rblk, ACC_W), lambda i: (0, i, 0)),
            full((1, 128)),
            pl.BlockSpec((rblk, 128), lambda i: (i, 0)),
            full((128, 128)), full((128, 1)), full((128, 1)),
        ],
        out_specs=[
            pl.BlockSpec((rblk, 128), lambda i: (i, 0)),
            pl.BlockSpec((rblk, 128), lambda i: (i, 0)),
            pl.BlockSpec((rblk, 1), lambda i: (i, 0)),
            pl.BlockSpec((rblk, 1), lambda i: (i, 0)),
        ],
        out_shape=[
            jax.ShapeDtypeStruct((n, 128), F32),
            jax.ShapeDtypeStruct((n, 128), F32),
            jax.ShapeDtypeStruct((n, 1), F32),
            jax.ShapeDtypeStruct((n, 1), F32),
        ],
    )(acc, bvec, run, w_next, asrc_n, adst_n)


def _gat_epi_last_call(acc, bvec, run):
    n = run.shape[0]
    rblk = 240 if n % 240 == 0 else 160
    gi = n // rblk
    full = lambda s: pl.BlockSpec(s, lambda i: (0,) * len(s))

    def body(acc_r, b_r, run_r, run_o):
        a = jnp.sum(acc_r[...], axis=0)
        x = _relu(a[:, :128] / (a[:, 128:129] + 1e-16) + b_r[...])
        run_o[...] = run_r[...] + x

    return pl.pallas_call(
        body,
        grid=(gi,),
        in_specs=[
            pl.BlockSpec((NW, rblk, ACC_W), lambda i: (0, i, 0)),
            full((1, 128)),
            pl.BlockSpec((rblk, 128), lambda i: (i, 0)),
        ],
        out_specs=pl.BlockSpec((rblk, 128), lambda i: (i, 0)),
        out_shape=jax.ShapeDtypeStruct((n, 128), F32),
    )(acc, bvec, run)


def _qkv_call(xg, amis, wq, bq, wk, bk, wv, bv, batch1c, amib_row):
    def body(xg_r, am_r, wq_r, bq_r, wk_r, bk_r, wv_r, bv_r, b1_r, ab_r,
             q_o, k_o, v_o, ep_o, wp_o):
        q = _dot(xg_r[...], wq_r[...]) + bq_r[...]
        q_o[...] = q
        k_o[...] = _dot(am_r[...], wk_r[...]) + bk_r[...]
        v_o[...] = _dot(am_r[...], wv_r[...]) + bv_r[...]
        scale = 1.0 / jnp.sqrt(128.0)
        ep_o[...] = _dot(q, bk_r[...].reshape(128, 1)) * scale
        gids = lax.broadcasted_iota(jnp.int32, (B, N3), 0)
        cnt = jnp.sum(jnp.where(ab_r[...] == gids, 1.0, 0.0), axis=1,
                      keepdims=True)
        maxp = jnp.max(cnt)
        pad_true = maxp - cnt
        nfree = jnp.float32(N3) - cnt
        w_pad = nfree * pad_true / jnp.maximum(nfree, 1.0)
        g2 = lax.broadcasted_iota(jnp.int32, (N1, B), 1)
        onehot = jnp.where(b1_r[...] == g2, 1.0, 0.0)
        wp_o[...] = _dot(onehot, w_pad)

    return pl.pallas_call(
        body,
        out_shape=[
            jax.ShapeDtypeStruct((N1, 128), F32),
            jax.ShapeDtypeStruct((N3, 128), F32),
            jax.ShapeDtypeStruct((N3, 128), F32),
            jax.ShapeDtypeStruct((N1, 1), F32),
            jax.ShapeDtypeStruct((N1, 1), F32),
        ],
    )(xg, amis, wq, bq, wk, bk, wv, bv, batch1c, amib_row)


def _cross_att_call(q, k, v, batch1c, amib_row, epad, wprow,
                    cafc_W, cafc_b, wv_b, ln_g, ln_b):
    blk = 128
    gi = N1 // blk

    def body(q_r, k_r, v_r, b1_r, ab_r, ep_r, wp_r, cw_r, cb_r, vb_r,
             lg_r, lb_r, out_r):
        scale = 1.0 / jnp.sqrt(128.0)
        e = lax.dot_general(q_r[...], k_r[...],
                            (((1,), (1,)), ((), ())),
                            preferred_element_type=F32) * scale
        mask = b1_r[...] == ab_r[...]
        em = jnp.where(mask, e, -1e30)
        ep = ep_r[...]
        m = jnp.maximum(jnp.max(em, axis=1, keepdims=True), ep)
        ex = jnp.exp(em - m)
        epw = wp_r[...] * jnp.exp(ep - m)
        s = jnp.sum(ex, axis=1, keepdims=True) + epw
        o = (_dot(ex, v_r[...]) + epw * vb_r[...]) / s
        o = _dot(o, cw_r[...]) + cb_r[...]
        mu = jnp.mean(o, axis=1, keepdims=True)
        var = jnp.mean((o - mu) ** 2, axis=1, keepdims=True)
        out_r[...] = (o - mu) / jnp.sqrt(var + 1e-5) * lg_r[...] + lb_r[...]

    full = lambda s: pl.BlockSpec(s, lambda i: (0,) * len(s))
    return pl.pallas_call(
        body,
        grid=(gi,),
        in_specs=[
            pl.BlockSpec((blk, 128), lambda i: (i, 0)),
            full((N3, 128)), full((N3, 128)),
            pl.BlockSpec((blk, 1), lambda i: (i, 0)),
            full((1, N3)),
            pl.BlockSpec((blk, 1), lambda i: (i, 0)),
            pl.BlockSpec((blk, 1), lambda i: (i, 0)),
            full((128, 128)), full((1, 128)), full((1, 128)),
            full((1, 128)), full((1, 128)),
        ],
        out_specs=pl.BlockSpec((blk, 128), lambda i: (i, 0)),
        out_shape=jax.ShapeDtypeStruct((N1, 128), F32),
    )(q, k, v, batch1c, amib_row, epad, wprow, cafc_W, cafc_b, wv_b,
      ln_g, ln_b)


def _node_head_call(xg, ami_att, fc0_W, fc0_b, fca_W, fca_b):
    def body(xg_r, aa_r, w_r, b_r, aw_r, ab_r, out_r):
        w = w_r[...]
        x = _relu(_dot(xg_r[...], w[:128, :]) + _dot(aa_r[...], w[128:, :])
                  + b_r[...])
        a = jnp.tanh(_dot(x, aw_r[...]) + ab_r[...])
        out_r[...] = x * a

    return pl.pallas_call(
        body,
        out_shape=jax.ShapeDtypeStruct((N1, 128), F32),
    )(xg, ami_att, fc0_W, fc0_b, fca_W, fca_b)


def _seg_reduce_call(x, batch1r):
    def body(x_r, b_r, sum_o, max_o, cnt_o):
        g = pl.program_id(0)
        msk = (b_r[...] == g).astype(F32)
        cnt_o[...] = jnp.sum(msk, axis=1, keepdims=True).reshape(1, 1, 1)
        mcol = msk.reshape(N1, 1)
        sum_o[...] = jnp.sum(x_r[...] * mcol, axis=0,
                             keepdims=True).reshape(1, 1, 128)
        max_o[...] = jnp.max(jnp.where(mcol > 0.0, x_r[...], -1e30),
                             axis=0, keepdims=True).reshape(1, 1, 128)

    full = lambda s: pl.BlockSpec(s, lambda i: (0,) * len(s))
    ssum, smax, cnt = pl.pallas_call(
        body,
        grid=(B,),
        in_specs=[full((N1, 128)), full((1, N1))],
        out_specs=[
            pl.BlockSpec((1, 1, 128), lambda i: (i, 0, 0)),
            pl.BlockSpec((1, 1, 128), lambda i: (i, 0, 0)),
            pl.BlockSpec((1, 1, 1), lambda i: (i, 0, 0)),
        ],
        out_shape=[
            jax.ShapeDtypeStruct((B, 1, 128), F32),
            jax.ShapeDtypeStruct((B, 1, 128), F32),
            jax.ShapeDtypeStruct((B, 1, 1), F32),
        ],
    )(x, batch1r)
    return ssum.reshape(B, 128), smax.reshape(B, 128), cnt.reshape(B, 1)


def _final_head_call(ssum, smax, cnt, cafc_b, ln_g, ln_b, fc0_W, fc0_b,
                     fca_W, fca_b, fc1_W, fc1_b, fc2_W, fc2_b,
                     fc3_W, fc3_b, out_W, out_b):
    def body(ss_r, sm_r, c_r, cb_r, lg_r, lb_r, w0_r, b0_r, aw_r, ab_r,
             w1_r, b1_r, w2_r, b2_r, w3_r, b3_r, wo_r, bo_r,
             out_o, h2_o):
        cb = cb_r[...]
        mu = jnp.mean(cb, axis=1, keepdims=True)
        var = jnp.mean((cb - mu) ** 2, axis=1, keepdims=True)
        pad_att = (cb - mu) / jnp.sqrt(var + 1e-5) * lg_r[...] + lb_r[...]
        pr = _relu(_dot(pad_att, w0_r[...][128:, :]) + b0_r[...])
        pr = pr * jnp.tanh(_dot(pr, aw_r[...]) + ab_r[...])
        cnt = c_r[...]
        maxl = jnp.max(cnt)
        mean_x = (ss_r[...] + (maxl - cnt) * pr) / maxl
        max_x = jnp.where(maxl > cnt, jnp.maximum(sm_r[...], pr), sm_r[...])
        xh = jnp.concatenate([mean_x, max_x], axis=1)
        h2 = _dot(xh, w1_r[...]) + b1_r[...]
        h2_o[...] = h2
        x = _relu(h2)
        x = _relu(_dot(x, w2_r[...]) + b2_r[...])
        x = _relu(_dot(x, w3_r[...]) + b3_r[...])
        out_o[...] = _dot(x, wo_r[...]) + bo_r[...]

    return pl.pallas_call(
        body,
        out_shape=[
            jax.ShapeDtypeStruct((B, 1), F32),
            jax.ShapeDtypeStruct((B, 512), F32),
        ],
    )(ssum, smax, cnt, cafc_b, ln_g, ln_b, fc0_W, fc0_b, fca_W, fca_b,
      fc1_W, fc1_b, fc2_W, fc2_b, fc3_W, fc3_b, out_W, out_b)


def _gat_chain(x0_run, h, a_s, a_d, src, dst, dstp, e_real, n_chunks, n,
               weights):
    n_pad = ((n + CHUNK + 127) // 128) * 128
    sc = _gat_sc_build(n, n_pad, n_chunks, e_real)
    zeros = jnp.zeros((n_pad, ACC_W), F32)
    run = x0_run
    for li in range(3):
        acc = sc(h, a_s.reshape(n), a_d.reshape(n), src, dst, dstp, zeros)
        b_l = weights[li][3]
        if li < 2:
            w_n, as_n, ad_n, _ = weights[li + 1]
            run, h, a_s, a_d = _gat_epi_call(
                acc, b_l.reshape(1, 128), run, w_n,
                as_n.reshape(128, 1), ad_n.reshape(128, 1))
        else:
            run = _gat_epi_last_call(acc, b_l.reshape(1, 128), run)
    return run


def kernel(x1, drug_intra, drug_edge_attr, batch1, x3, ami_intra, ami_dis,
           ami_batch, ami_dis_li, w1, b1, w2, b2,
           fc00_W, fc00_b, fc01_W, fc01_b, fc02_W, fc02_b, fc03_W, fc03_b,
           conv1_W, conv1_asrc, conv1_adst, conv1_b,
           conv2_W, conv2_asrc, conv2_adst, conv2_b,
           conv3_W, conv3_asrc, conv3_adst, conv3_b,
           conv4_W, conv4_asrc, conv4_adst, conv4_b,
           conv5_W, conv5_asrc, conv5_adst, conv5_b,
           conv6_W, conv6_asrc, conv6_adst, conv6_b,
           wq_W, wq_b, wk_W, wk_b, wv_W, wv_b, cafc_W, cafc_b,
           ln_g, ln_b, fc0_W, fc0_b, fc_att_W, fc_att_b,
           fc1_W, fc1_b, fc2_W, fc2_b, fc3_W, fc3_b, out_W, out_b):
    w02p = jnp.pad(fc02_W, ((0, 0), (0, 1)))
    b02p = jnp.pad(fc02_b, (0, 1)).reshape(1, 128)
    src3, dst3, dstp3, er3, nc3 = _pad_edges(ami_intra, N3)
    src1, dst1, dstp1, er1, nc1 = _pad_edges(drug_intra, N1)
    b1c = batch1.astype(jnp.int32).reshape(N1, 1)
    b1r = batch1.astype(jnp.int32).reshape(1, N1)
    abr = ami_batch.astype(jnp.int32).reshape(1, N3)

    ami0, h4, as4, ad4 = _prot_mlp_call(
        x3, ami_dis_li.reshape(N3, 1),
        fc00_W, fc00_b.reshape(1, 1024),
        fc01_W, fc01_b.reshape(1, 512),
        w02p, b02p,
        conv4_W, conv4_asrc.reshape(128, 1), conv4_adst.reshape(128, 1))
    amis = _gat_chain(
        ami0, h4, as4, ad4, src3, dst3, dstp3, er3, nc3, N3,
        [(conv4_W, conv4_asrc, conv4_adst, conv4_b),
         (conv5_W, conv5_asrc, conv5_adst, conv5_b),
         (conv6_W, conv6_asrc, conv6_adst, conv6_b)])

    x10, h1, as1, ad1 = _drug_mlp_call(
        x1, w1, b1.reshape(1, 128), w2, b2.reshape(1, 64),
        fc03_W, fc03_b.reshape(1, 128),
        conv1_W, conv1_asrc.reshape(128, 1), conv1_adst.reshape(128, 1))
    xg = _gat_chain(
        x10, h1, as1, ad1, src1, dst1, dstp1, er1, nc1, N1,
        [(conv1_W, conv1_asrc, conv1_adst, conv1_b),
         (conv2_W, conv2_asrc, conv2_adst, conv2_b),
         (conv3_W, conv3_asrc, conv3_adst, conv3_b)])

    q, k, v, epad, wprow = _qkv_call(
        xg, amis, wq_W, wq_b.reshape(1, 128), wk_W, wk_b.reshape(1, 128),
        wv_W, wv_b.reshape(1, 128), b1c, abr)
    ami_att = _cross_att_call(
        q, k, v, b1c, abr, epad, wprow, cafc_W, cafc_b.reshape(1, 128),
        wv_b.reshape(1, 128), ln_g.reshape(1, 128), ln_b.reshape(1, 128))

    xnode = _node_head_call(xg, ami_att, fc0_W, fc0_b.reshape(1, 128),
                            fc_att_W, fc_att_b.reshape(1, 1))
    ssum, smax, cnt = _seg_reduce_call(xnode, b1r)
    out, h2 = _final_head_call(
        ssum, smax, cnt, cafc_b.reshape(1, 128), ln_g.reshape(1, 128),
        ln_b.reshape(1, 128), fc0_W, fc0_b.reshape(1, 128),
        fc_att_W, fc_att_b.reshape(1, 1),
        fc1_W, fc1_b.reshape(1, 512), fc2_W, fc2_b.reshape(1, 256),
        fc3_W, fc3_b.reshape(1, 128), out_W, out_b.reshape(1, 1))
    return (out, h2)
